# Initial kernel scaffold; baseline (speedup 1.0000x reference)
import functools

import jax
import jax.numpy as jnp
from jax import lax
from jax.experimental import pallas as pl
from jax.experimental.pallas import tpu as pltpu
from jax.experimental.pallas import tpu_sc as plsc

LANES = 16
NC = 2
NS = 16
NW = NC * NS

RPG = 128
K = 10
CHUNK = K * RPG
D = 32


def _emb_body(idx_hbm, tab_hbm, out_hbm, idx_v, rows_v, fmask_v, sem):
    wid = lax.axis_index("s") * NC + lax.axis_index("c")
    n_rows = idx_hbm.shape[0] * RPG
    rows_per_w = n_rows // NW
    n_chunks = rows_per_w // CHUNK
    row0 = wid * rows_per_w

    def chunk_body(c, carry):
        rbase = row0 + c * CHUNK
        gbase = rbase // RPG
        pltpu.sync_copy(idx_hbm.at[pl.ds(gbase, K), :], idx_v)
        cps = [
            pltpu.async_copy(
                tab_hbm.at[idx_v.at[g]],
                rows_v.at[pl.ds(g * RPG, RPG), :],
                sem,
            )
            for g in range(K)
        ]
        for cp in cps:
            cp.wait()

        def mask_body(g, carry2):
            v = idx_v[g // 8, pl.ds((g % 8) * LANES, LANES)]
            fmask_v[...] = jnp.where(v == 0, 0.0, 1.0).astype(jnp.float32)
            for r in range(LANES):
                m = plsc.load_gather(
                    fmask_v, [jnp.full((LANES,), r, jnp.int32)]
                )
                rr = g * LANES + r
                rows_v[rr, pl.ds(0, LANES)] = rows_v[rr, pl.ds(0, LANES)] * m
                rows_v[rr, pl.ds(LANES, LANES)] = (
                    rows_v[rr, pl.ds(LANES, LANES)] * m
                )
            return carry2

        lax.fori_loop(0, CHUNK // LANES, mask_body, 0)
        pltpu.sync_copy(rows_v, out_hbm.at[pl.ds(rbase, CHUNK), :])
        return carry

    lax.fori_loop(0, n_chunks, chunk_body, 0)


def kernel(x, weight):
    b, s = x.shape
    n_rows = b * s
    idx = x.reshape(n_rows // RPG, RPG).astype(jnp.int32)

    mesh = plsc.VectorSubcoreMesh(core_axis_name="c", subcore_axis_name="s")
    fn = functools.partial(
        pl.kernel,
        mesh=mesh,
        out_type=jax.ShapeDtypeStruct((n_rows, D), jnp.float32),
        scratch_types=[
            pltpu.VMEM((K, RPG), jnp.int32),
            pltpu.VMEM((CHUNK, D), jnp.float32),
            pltpu.VMEM((LANES,), jnp.float32),
            pltpu.SemaphoreType.DMA,
        ],
    )(_emb_body)
    out = fn(idx, weight)
    return out.reshape(b, s, D)


# speedup vs baseline: 1.4824x; 1.4824x over previous
import functools

import jax
import jax.numpy as jnp
from jax import lax
from jax.experimental import pallas as pl
from jax.experimental.pallas import tpu as pltpu
from jax.experimental.pallas import tpu_sc as plsc

LANES = 16
NC = 2
NS = 16
NW = NC * NS

RPG = 128
K = 8
CHUNK = K * RPG
D = 32


def _emb_body(idx_hbm, tab_hbm, out_hbm, idx_v, rows_v, sem):
    wid = lax.axis_index("s") * NC + lax.axis_index("c")
    n_rows = idx_hbm.shape[0] * RPG
    rows_per_w = n_rows // NW
    n_chunks = rows_per_w // CHUNK
    row0 = wid * rows_per_w

    def chunk_body(c, carry):
        rbase = row0 + c * CHUNK
        gbase = pl.multiple_of(rbase // RPG, 8)
        pltpu.sync_copy(idx_hbm.at[pl.ds(gbase, K), :], idx_v)
        cps = [
            pltpu.async_copy(
                tab_hbm.at[idx_v.at[g]],
                rows_v.at[pl.ds(g * RPG, RPG), :],
                sem,
            )
            for g in range(K)
        ]
        for cp in cps:
            cp.wait()

        dnums = lax.GatherDimensionNumbers(
            offset_dims=(), collapsed_slice_dims=(0,), start_index_map=(0,)
        )

        def mask_body(g, carry2):
            v = idx_v[g // 8, pl.ds((g % 8) * LANES, LANES)]
            fmask = jnp.where(v == 0, 0.0, 1.0).astype(jnp.float32)
            for r in range(LANES):
                m = lax.gather(
                    fmask,
                    jnp.full((LANES, 1), r, jnp.int32),
                    dnums,
                    (1,),
                    mode=lax.GatherScatterMode.PROMISE_IN_BOUNDS,
                )
                rr = g * LANES + r
                rows_v[rr, pl.ds(0, LANES)] = rows_v[rr, pl.ds(0, LANES)] * m
                rows_v[rr, pl.ds(LANES, LANES)] = (
                    rows_v[rr, pl.ds(LANES, LANES)] * m
                )
            return carry2

        lax.fori_loop(0, CHUNK // LANES, mask_body, 0)
        pltpu.sync_copy(rows_v, out_hbm.at[pl.ds(rbase, CHUNK), :])
        return carry

    lax.fori_loop(0, n_chunks, chunk_body, 0)


def kernel(x, weight):
    b, s = x.shape
    n_rows = b * s
    idx = x.reshape(n_rows // RPG, RPG).astype(jnp.int32)

    mesh = plsc.VectorSubcoreMesh(core_axis_name="c", subcore_axis_name="s")
    fn = functools.partial(
        pl.kernel,
        mesh=mesh,
        out_type=jax.ShapeDtypeStruct((n_rows, D), jnp.float32),
        scratch_types=[
            pltpu.VMEM((K, RPG), jnp.int32),
            pltpu.VMEM((CHUNK, D), jnp.float32),
            pltpu.SemaphoreType.DMA,
        ],
        compiler_params=pltpu.CompilerParams(use_tc_tiling_on_sc=False),
    )(_emb_body)
    out = fn(idx, weight)
    return out.reshape(b, s, D)


# speedup vs baseline: 1.5694x; 1.0587x over previous
import functools

import jax
import jax.numpy as jnp
from jax import lax
from jax.experimental import pallas as pl
from jax.experimental.pallas import tpu as pltpu
from jax.experimental.pallas import tpu_sc as plsc

LANES = 16
NC = 2
NS = 16
NW = NC * NS

RPG = 128
K = 10
CHUNK = K * RPG
D = 32

_DNUMS = lax.GatherDimensionNumbers(
    offset_dims=(), collapsed_slice_dims=(0,), start_index_map=(0,)
)


def _fix_padding(idx_v, rows_v):
    gpr = RPG // LANES

    def detect_body(g, acc):
        v = idx_v[g // gpr, pl.ds((g % gpr) * LANES, LANES)]
        return jnp.minimum(acc, v)

    min_idx = lax.fori_loop(
        0,
        CHUNK // LANES,
        detect_body,
        jnp.full((LANES,), 1, jnp.int32),
        unroll=8,
    )
    n_pad = jnp.min(min_idx)

    @pl.when(n_pad == 0)
    def _():
        def mask_body(g, carry):
            v = idx_v[g // gpr, pl.ds((g % gpr) * LANES, LANES)]
            fmask = jnp.where(v == 0, 0.0, 1.0).astype(jnp.float32)
            for r in range(LANES):
                m = lax.gather(
                    fmask,
                    jnp.full((LANES, 1), r, jnp.int32),
                    _DNUMS,
                    (1,),
                    mode=lax.GatherScatterMode.PROMISE_IN_BOUNDS,
                )
                rr = g * LANES + r
                rows_v[rr, pl.ds(0, LANES)] = rows_v[rr, pl.ds(0, LANES)] * m
                rows_v[rr, pl.ds(LANES, LANES)] = (
                    rows_v[rr, pl.ds(LANES, LANES)] * m
                )
            return carry

        lax.fori_loop(0, CHUNK // LANES, mask_body, 0)


def _emb_body(idx_hbm, tab_hbm, out_hbm,
              idx0_v, idx1_v, rows0_v, rows1_v,
              sem_g0, sem_g1, sem_o0, sem_o1):
    wid = lax.axis_index("s") * NC + lax.axis_index("c")
    n_rows = idx_hbm.shape[0] * RPG
    rows_per_w = n_rows // NW
    n_chunks = rows_per_w // CHUNK
    row0 = pl.multiple_of(wid * rows_per_w, 8)

    idx_bufs = (idx0_v, idx1_v)
    rows_bufs = (rows0_v, rows1_v)
    sems_g = (sem_g0, sem_g1)
    sems_o = (sem_o0, sem_o1)

    def stage_and_fire(c, b):
        gbase = pl.multiple_of((row0 + c * CHUNK) // RPG, 2)
        pltpu.sync_copy(idx_hbm.at[pl.ds(gbase, K), :], idx_bufs[b])
        for g in range(K):
            pltpu.async_copy(
                tab_hbm.at[idx_bufs[b].at[g]],
                rows_bufs[b].at[pl.ds(g * RPG, RPG), :],
                sems_g[b],
            )

    def drain_gathers(c, b):
        rbase = pl.multiple_of(row0 + c * CHUNK, 8)
        pltpu.make_async_copy(
            out_hbm.at[pl.ds(rbase, CHUNK), :], rows_bufs[b], sems_g[b]
        ).wait()

    def fire_out(c, b):
        rbase = pl.multiple_of(row0 + c * CHUNK, 8)
        pltpu.async_copy(
            rows_bufs[b], out_hbm.at[pl.ds(rbase, CHUNK), :], sems_o[b]
        )

    def drain_out(c, b):
        rbase = pl.multiple_of(row0 + c * CHUNK, 8)
        pltpu.make_async_copy(
            rows_bufs[b], out_hbm.at[pl.ds(rbase, CHUNK), :], sems_o[b]
        ).wait()

    def chunk_step(c, b, p):
        ob = 1 - b

        @pl.when(c + 1 < n_chunks)
        def _():
            @pl.when(p > 0)
            def _():
                drain_out(c - 1, ob)

            stage_and_fire(c + 1, ob)

        drain_gathers(c, b)
        _fix_padding(idx_bufs[b], rows_bufs[b])
        fire_out(c, b)

    stage_and_fire(0, 0)

    def pair_body(p, carry):
        chunk_step(2 * p, 0, p)
        chunk_step(2 * p + 1, 1, p + 1)
        return carry

    lax.fori_loop(0, n_chunks // 2, pair_body, 0)
    drain_out(n_chunks - 2, 0)
    drain_out(n_chunks - 1, 1)


def kernel(x, weight):
    b, s = x.shape
    n_rows = b * s
    idx = x.reshape(n_rows // RPG, RPG).astype(jnp.int32)

    mesh = plsc.VectorSubcoreMesh(core_axis_name="c", subcore_axis_name="s")
    fn = functools.partial(
        pl.kernel,
        mesh=mesh,
        out_type=jax.ShapeDtypeStruct((n_rows, D), jnp.float32),
        scratch_types=[
            pltpu.VMEM((K, RPG), jnp.int32),
            pltpu.VMEM((K, RPG), jnp.int32),
            pltpu.VMEM((CHUNK, D), jnp.float32),
            pltpu.VMEM((CHUNK, D), jnp.float32),
            pltpu.SemaphoreType.DMA,
            pltpu.SemaphoreType.DMA,
            pltpu.SemaphoreType.DMA,
            pltpu.SemaphoreType.DMA,
        ],
        compiler_params=pltpu.CompilerParams(
            use_tc_tiling_on_sc=False, needs_layout_passes=False
        ),
    )(_emb_body)
    out = fn(idx, weight)
    return out.reshape(b, s, D)


# speedup vs baseline: 1.5704x; 1.0007x over previous
import functools

import jax
import jax.numpy as jnp
from jax import lax
from jax.experimental import pallas as pl
from jax.experimental.pallas import tpu as pltpu
from jax.experimental.pallas import tpu_sc as plsc

LANES = 16
NC = 2
NS = 16
NW = NC * NS

RPG = 128
K = 10
CHUNK = K * RPG
D = 32

_DNUMS = lax.GatherDimensionNumbers(
    offset_dims=(), collapsed_slice_dims=(0,), start_index_map=(0,)
)


def _fix_padding(idx_v, rows_v):
    gpr = RPG // LANES

    def detect_body(g, acc):
        v = idx_v[pl.ds(g * LANES, LANES)]
        return jnp.minimum(acc, v)

    min_idx = lax.fori_loop(
        0,
        CHUNK // LANES,
        detect_body,
        jnp.full((LANES,), 1, jnp.int32),
        unroll=8,
    )
    n_pad = jnp.min(min_idx)

    @pl.when(n_pad == 0)
    def _():
        def mask_body(g, carry):
            v = idx_v[pl.ds(g * LANES, LANES)]
            fmask = jnp.where(v == 0, 0.0, 1.0).astype(jnp.float32)
            for r in range(LANES):
                m = lax.gather(
                    fmask,
                    jnp.full((LANES, 1), r, jnp.int32),
                    _DNUMS,
                    (1,),
                    mode=lax.GatherScatterMode.PROMISE_IN_BOUNDS,
                )
                rr = g * LANES + r
                rows_v[rr, pl.ds(0, LANES)] = rows_v[rr, pl.ds(0, LANES)] * m
                rows_v[rr, pl.ds(LANES, LANES)] = (
                    rows_v[rr, pl.ds(LANES, LANES)] * m
                )
            return carry

        lax.fori_loop(0, CHUNK // LANES, mask_body, 0)


def _emb_body(idx_hbm, tab_hbm, out_hbm,
              idx0_v, idx1_v, rows0_v, rows1_v,
              sem_g0, sem_g1, sem_o0, sem_o1):
    wid = lax.axis_index("s") * NC + lax.axis_index("c")
    n_rows = idx_hbm.shape[0]
    rows_per_w = n_rows // NW
    n_chunks = rows_per_w // CHUNK
    row0 = pl.multiple_of(wid * rows_per_w, 8)

    idx_bufs = (idx0_v, idx1_v)
    rows_bufs = (rows0_v, rows1_v)
    sems_g = (sem_g0, sem_g1)
    sems_o = (sem_o0, sem_o1)

    def stage_and_fire(c, b):
        rbase = pl.multiple_of(row0 + c * CHUNK, 8)
        pltpu.sync_copy(idx_hbm.at[pl.ds(rbase, CHUNK)], idx_bufs[b])
        pltpu.async_copy(tab_hbm.at[idx_bufs[b]], rows_bufs[b], sems_g[b])

    def drain_gathers(c, b):
        rbase = pl.multiple_of(row0 + c * CHUNK, 8)
        pltpu.make_async_copy(
            out_hbm.at[pl.ds(rbase, CHUNK), :], rows_bufs[b], sems_g[b]
        ).wait()

    def fire_out(c, b):
        rbase = pl.multiple_of(row0 + c * CHUNK, 8)
        pltpu.async_copy(
            rows_bufs[b], out_hbm.at[pl.ds(rbase, CHUNK), :], sems_o[b]
        )

    def drain_out(c, b):
        rbase = pl.multiple_of(row0 + c * CHUNK, 8)
        pltpu.make_async_copy(
            rows_bufs[b], out_hbm.at[pl.ds(rbase, CHUNK), :], sems_o[b]
        ).wait()

    def chunk_step(c, b, p):
        ob = 1 - b

        @pl.when(c + 1 < n_chunks)
        def _():
            @pl.when(p > 0)
            def _():
                drain_out(c - 1, ob)

            stage_and_fire(c + 1, ob)

        drain_gathers(c, b)
        _fix_padding(idx_bufs[b], rows_bufs[b])
        fire_out(c, b)

    stage_and_fire(0, 0)

    def pair_body(p, carry):
        chunk_step(2 * p, 0, p)
        chunk_step(2 * p + 1, 1, p + 1)
        return carry

    lax.fori_loop(0, n_chunks // 2, pair_body, 0)
    drain_out(n_chunks - 2, 0)
    drain_out(n_chunks - 1, 1)


def kernel(x, weight):
    b, s = x.shape
    n_rows = b * s
    idx = x.reshape(n_rows).astype(jnp.int32)

    mesh = plsc.VectorSubcoreMesh(core_axis_name="c", subcore_axis_name="s")
    fn = functools.partial(
        pl.kernel,
        mesh=mesh,
        out_type=jax.ShapeDtypeStruct((n_rows, D), jnp.float32),
        scratch_types=[
            pltpu.VMEM((CHUNK,), jnp.int32),
            pltpu.VMEM((CHUNK,), jnp.int32),
            pltpu.VMEM((CHUNK, D), jnp.float32),
            pltpu.VMEM((CHUNK, D), jnp.float32),
            pltpu.SemaphoreType.DMA,
            pltpu.SemaphoreType.DMA,
            pltpu.SemaphoreType.DMA,
            pltpu.SemaphoreType.DMA,
        ],
        compiler_params=pltpu.CompilerParams(
            use_tc_tiling_on_sc=False, needs_layout_passes=False
        ),
    )(_emb_body)
    out = fn(idx, weight)
    return out.reshape(b, s, D)


# speedup vs baseline: 2.1362x; 1.3602x over previous
import functools

import jax
import jax.numpy as jnp
from jax import lax
from jax.experimental import pallas as pl
from jax.experimental.pallas import tpu as pltpu
from jax.experimental.pallas import tpu_sc as plsc

LANES = 16
NC = 2
NS = 16
NW = NC * NS

RPG = 128
K = 10
CHUNK = K * RPG
D = 32

_DNUMS = lax.GatherDimensionNumbers(
    offset_dims=(), collapsed_slice_dims=(0,), start_index_map=(0,)
)


def _fix_padding(idx_v, rows_v):
    gpr = RPG // LANES

    def detect_body(g, acc):
        v = idx_v[pl.ds(g * LANES, LANES)]
        return jnp.minimum(acc, v)

    min_idx = lax.fori_loop(
        0,
        CHUNK // LANES,
        detect_body,
        jnp.full((LANES,), 1, jnp.int32),
        unroll=8,
    )
    n_pad = jnp.min(min_idx)

    @pl.when(n_pad == 0)
    def _():
        def mask_body(g, carry):
            v = idx_v[pl.ds(g * LANES, LANES)]
            fmask = jnp.where(v == 0, 0.0, 1.0).astype(jnp.float32)
            for r in range(LANES):
                m = lax.gather(
                    fmask,
                    jnp.full((LANES, 1), r, jnp.int32),
                    _DNUMS,
                    (1,),
                    mode=lax.GatherScatterMode.PROMISE_IN_BOUNDS,
                )
                rr = g * LANES + r
                rows_v[rr, pl.ds(0, LANES)] = rows_v[rr, pl.ds(0, LANES)] * m
                rows_v[rr, pl.ds(LANES, LANES)] = (
                    rows_v[rr, pl.ds(LANES, LANES)] * m
                )
            return carry

        lax.fori_loop(0, CHUNK // LANES, mask_body, 0)


def _emb_body(idx_hbm, tab_hbm, out_hbm,
              idx0_v, idx1_v, rows0_v, rows1_v,
              sem_g0, sem_g1, sem_o0, sem_o1):
    wid = lax.axis_index("s") * NC + lax.axis_index("c")
    n_rows = idx_hbm.shape[0]
    rows_per_w = n_rows // NW
    n_chunks = rows_per_w // CHUNK
    row0 = pl.multiple_of(wid * rows_per_w, 8)

    idx_bufs = (idx0_v, idx1_v)
    rows_bufs = (rows0_v, rows1_v)
    sems_g = (sem_g0, sem_g1)
    sems_o = (sem_o0, sem_o1)

    def stage_and_fire(c, b):
        rbase = pl.multiple_of(row0 + c * CHUNK, 8)
        pltpu.sync_copy(idx_hbm.at[pl.ds(rbase, CHUNK)], idx_bufs[b])
        pltpu.async_copy(tab_hbm.at[idx_bufs[b]], rows_bufs[b], sems_g[b])

    def drain_gathers(c, b):
        rbase = pl.multiple_of(row0 + c * CHUNK, 8)
        pltpu.make_async_copy(
            out_hbm.at[pl.ds(rbase, CHUNK), pl.ds(0, D)],
            rows_bufs[b],
            sems_g[b],
        ).wait()

    def fire_out(c, b):
        rbase = pl.multiple_of(row0 + c * CHUNK, 8)
        pltpu.async_copy(
            rows_bufs[b],
            out_hbm.at[pl.ds(rbase, CHUNK), pl.ds(0, D)],
            sems_o[b],
        )

    def drain_out(c, b):
        rbase = pl.multiple_of(row0 + c * CHUNK, 8)
        pltpu.make_async_copy(
            rows_bufs[b],
            out_hbm.at[pl.ds(rbase, CHUNK), pl.ds(0, D)],
            sems_o[b],
        ).wait()

    def chunk_step(c, b, p):
        ob = 1 - b

        @pl.when(c + 1 < n_chunks)
        def _():
            @pl.when(p > 0)
            def _():
                drain_out(c - 1, ob)

            stage_and_fire(c + 1, ob)

        drain_gathers(c, b)
        _fix_padding(idx_bufs[b], rows_bufs[b])
        fire_out(c, b)

    stage_and_fire(0, 0)

    def pair_body(p, carry):
        chunk_step(2 * p, 0, p)
        chunk_step(2 * p + 1, 1, p + 1)
        return carry

    lax.fori_loop(0, n_chunks // 2, pair_body, 0)
    drain_out(n_chunks - 2, 0)
    drain_out(n_chunks - 1, 1)


def kernel(x, weight):
    b, s = x.shape
    n_rows = b * s
    idx = x.reshape(n_rows).astype(jnp.int32)

    mesh = plsc.VectorSubcoreMesh(core_axis_name="c", subcore_axis_name="s")
    fn = functools.partial(
        pl.kernel,
        mesh=mesh,
        out_type=jax.ShapeDtypeStruct((n_rows, 128), jnp.float32),
        scratch_types=[
            pltpu.VMEM((CHUNK,), jnp.int32),
            pltpu.VMEM((CHUNK,), jnp.int32),
            pltpu.VMEM((CHUNK, D), jnp.float32),
            pltpu.VMEM((CHUNK, D), jnp.float32),
            pltpu.SemaphoreType.DMA,
            pltpu.SemaphoreType.DMA,
            pltpu.SemaphoreType.DMA,
            pltpu.SemaphoreType.DMA,
        ],
        compiler_params=pltpu.CompilerParams(
            use_tc_tiling_on_sc=False, needs_layout_passes=False
        ),
    )(_emb_body)
    out = fn(idx, weight)
    return out[:, :D].reshape(b, s, D)
